# baseline (speedup 1.0000x reference)
import functools

import jax
import jax.numpy as jnp
from jax import lax
from jax.experimental import pallas as pl
from jax.experimental.pallas import tpu as pltpu
from jax.experimental.pallas import tpu_sc as plsc

BATCH = 4096
SEQ = 200
DIM = 64
CHUNK = 100
IDW = 104
NROWS = BATCH * SEQ // CHUNK
NW = 32
CPW = NROWS // NW
BLK = 16
NPB = 3
AH = 2
L = 16
GOFF = (0, 16, 32, 48, 64, 80, 88)
VOCAB_PAIRS = 500000


def _build():
    mesh = plsc.VectorSubcoreMesh(core_axis_name="c", subcore_axis_name="s")

    @functools.partial(
        pl.kernel,
        mesh=mesh,
        out_type=jax.ShapeDtypeStruct((BATCH, SEQ, DIM), jnp.float32),
        scratch_types=[
            pltpu.VMEM((BLK, IDW), jnp.int32),
            pltpu.VMEM((SEQ, DIM), jnp.float32),
            [pltpu.VMEM((IDW,), jnp.int32) for _ in range(NPB)],
            [pltpu.VMEM((IDW,), jnp.int32) for _ in range(NPB)],
            [pltpu.VMEM((IDW, 2 * DIM), jnp.float32) for _ in range(NPB)],
            pltpu.VMEM((SEQ, DIM), jnp.float32),
            [pltpu.SemaphoreType.DMA for _ in range(NPB)],
        ],
        compiler_params=pltpu.CompilerParams(
            use_tc_tiling_on_sc=True, needs_layout_passes=False),
    )
    def emb(ids_hbm, table_hbm, pos_hbm, out_hbm, idsblk, pos_v, idx2_v,
            parw_v, pair_v, ob, gsem):
        wid = lax.axis_index("s") * 2 + lax.axis_index("c")
        base = wid * CPW
        sbase = base // 2
        pltpu.sync_copy(pos_hbm, pos_v)

        def prep(jj):
            q = jj % NPB
            for off in GOFF:
                t = idsblk[jj, pl.ds(off, L)]
                idx2_v[q][pl.ds(off, L)] = jnp.minimum(
                    lax.shift_right_logical(t, 1), VOCAB_PAIRS - 1)
                parw_v[q][pl.ds(off, L)] = jnp.bitwise_and(t, 1)
            pltpu.async_copy(table_hbm.at[idx2_v[q]], pair_v[q], gsem[q])

        def wait_gather(q):
            pltpu.make_async_copy(
                table_hbm.at[idx2_v[q]], pair_v[q], gsem[q]).wait()

        def blk_body(blk, carry):
            pltpu.sync_copy(ids_hbm.at[pl.ds(base + BLK * blk, BLK)], idsblk)
            for jj in range(AH):
                prep(jj)
            for jj in range(BLK):
                q = jj % NPB
                p0 = (jj & 1) * CHUNK
                if jj + AH < BLK:
                    prep(jj + AH)
                wait_gather(q)

                pv = pair_v[q]
                parw = parw_v[q]

                @plsc.parallel_loop(0, CHUNK, step=2, unroll=2)
                def tok_body(j):
                    for r in range(2):
                        jsplat = jnp.full((L,), j + r, jnp.int32)
                        m = plsc.load_gather(parw, (jsplat,)) > 0
                        for k in range(DIM // L):
                            lo = pv[j + r, pl.ds(L * k, L)]
                            hi = pv[j + r, pl.ds(DIM + L * k, L)]
                            ob[p0 + j + r, pl.ds(L * k, L)] = (
                                jnp.where(m, hi, lo)
                                + pos_v[p0 + j + r, pl.ds(L * k, L)])

                if p0 == CHUNK:
                    seq = sbase + (BLK // 2) * blk + (jj >> 1)
                    pltpu.sync_copy(ob, out_hbm.at[seq])
            return carry

        lax.fori_loop(0, CPW // BLK, blk_body, 0)

    return emb


_emb = _build()


def kernel(token_ids, token_table, pos_table):
    ids = token_ids.astype(jnp.int32).reshape(NROWS, CHUNK)
    ids = jnp.pad(ids, ((0, 0), (0, IDW - CHUNK)))
    table_p = token_table.reshape(VOCAB_PAIRS, 2 * DIM)
    return _emb(ids, table_p, pos_table)
